# Initial kernel scaffold; baseline (speedup 1.0000x reference)
import jax
import jax.numpy as jnp
from jax.experimental import pallas as pl


def kernel(prediction, target):
    raise NotImplementedError("write your pallas kernel here")


# speedup vs baseline: 28.6329x; 28.6329x over previous
import jax
import jax.numpy as jnp
from jax.experimental import pallas as pl
from jax.experimental.pallas import tpu as pltpu

_LAMBDA_SSI = 0.5
_TOP_K_MASKING = 0.1
_EPS = 1e-06

_B = 16
_N = 384 * 384
_C = 18432
_NCHUNK = _N // _C


def _body(p_ref, t_ref, out_ref, d_scr, u_scr, nv_scr):
    i = pl.program_id(0)

    p = p_ref[...]
    t = t_ref[...]
    mask = (t > _EPS)
    d = (jnp.log(jnp.maximum(p, _EPS)) - jnp.log(jnp.maximum(t, _EPS)))
    d = jnp.where(mask, d, 0.0)
    u = jax.lax.bitcast_convert_type(jnp.abs(d), jnp.int32)

    d_scr[:, pl.ds(i * _C, _C)] = d
    u_scr[:, pl.ds(i * _C, _C)] = u

    @pl.when(i == 0)
    def _init():
        nv_scr[...] = jnp.zeros_like(nv_scr)

    nv_scr[...] += jnp.sum(mask.astype(jnp.float32), axis=1, keepdims=True)

    @pl.when(i == _NCHUNK - 1)
    def _final():
        nv = nv_scr[...]
        k = (nv * _TOP_K_MASKING).astype(jnp.int32)
        k = jnp.minimum(k, _N - 1)

        u_all = u_scr[...]

        def step(_, carry):
            lo, hi = carry
            mid = lo + ((hi - lo) >> 1)
            cnt = jnp.sum((u_all > mid).astype(jnp.int32), axis=1,
                          keepdims=True)
            ok = cnt <= k
            return jnp.where(ok, lo, mid + 1), jnp.where(ok, mid, hi)

        lo0 = jnp.zeros((_B, 1), jnp.int32)
        hi0 = jnp.full((_B, 1), 0x7F800000, jnp.int32)
        _, thr = jax.lax.fori_loop(0, 31, step, (lo0, hi0))

        d_all = d_scr[...]
        keep = (u_all < thr).astype(jnp.float32)
        s = jnp.sum(d_all * keep, axis=1, keepdims=True)
        ss = jnp.sum(d_all * d_all * keep, axis=1, keepdims=True)
        cex = jnp.sum((u_all >= thr).astype(jnp.float32), axis=1,
                      keepdims=True)
        n = jnp.maximum(nv - cex, 1.0)

        term1 = ss / n
        term2 = _LAMBDA_SSI * (s * s) / (n * n)
        row = jnp.sqrt(jnp.maximum(term1 - term2, _EPS))
        out_ref[...] = jnp.mean(row).reshape(1, 1)


def kernel(prediction, target):
    p = prediction.reshape(_B, _N)
    t = target.reshape(_B, _N)
    out = pl.pallas_call(
        _body,
        grid=(_NCHUNK,),
        in_specs=[
            pl.BlockSpec((_B, _C), lambda i: (0, i)),
            pl.BlockSpec((_B, _C), lambda i: (0, i)),
        ],
        out_specs=pl.BlockSpec((1, 1), lambda i: (0, 0)),
        out_shape=jax.ShapeDtypeStruct((1, 1), jnp.float32),
        scratch_shapes=[
            pltpu.VMEM((_B, _N), jnp.float32),
            pltpu.VMEM((_B, _N), jnp.int32),
            pltpu.VMEM((_B, 1), jnp.float32),
        ],
    )(p, t)
    return out[0, 0]
